# baseline (speedup 1.0000x reference)
import functools

import jax
import jax.numpy as jnp
from jax import lax
from jax.experimental import pallas as pl
from jax.experimental.pallas import tpu as pltpu
from jax.experimental.pallas import tpu_sc as plsc

VOCAB = 1_000_000
DIM = 64
WORLD_SIZE = 1
RANK = 0
BLOCK = (VOCAB + WORLD_SIZE - 1) // WORLD_SIZE
MIN_ID = RANK * BLOCK
MAX_ID = min(VOCAB, (RANK + 1) * BLOCK)
NULL_IDX = MAX_ID - MIN_ID

NC = 2
NS = 16
NW = NC * NS
LANES = 16

TOKENS = 4096 * 50
BPW = TOKENS // NW
CHUNK = 128
NCHUNK = BPW // CHUNK
GROUP = 5
NGROUP = NCHUNK // GROUP


def _body(idx_hbm, w_hbm, out_hbm, idx_v, buf, sem):
    wid = lax.axis_index("s") * NC + lax.axis_index("c")
    pltpu.sync_copy(idx_hbm.at[wid], idx_v)

    def remap(t, carry):
        row = t // (CHUNK // LANES)
        col = (t % (CHUNK // LANES)) * LANES
        v = idx_v[row, pl.ds(col, LANES)]
        oob = (v < MIN_ID) | (v >= MAX_ID)
        idx_v[row, pl.ds(col, LANES)] = jnp.where(oob, NULL_IDX, v - MIN_ID)
        return carry

    lax.fori_loop(0, BPW // LANES, remap, None)

    base = wid * BPW

    def group(g, carry):
        handles = []
        for b in range(GROUP):
            j = g * GROUP + b
            h = pltpu.async_copy(
                w_hbm.at[idx_v.at[j]], buf.at[pl.ds(b * CHUNK, CHUNK)], sem
            )
            handles.append(h)
        for h in handles:
            h.wait()
        pltpu.sync_copy(
            buf, out_hbm.at[pl.ds(base + g * (GROUP * CHUNK), GROUP * CHUNK)]
        )
        return carry

    lax.fori_loop(0, NGROUP, group, None)


@jax.jit
def kernel(input, weight):
    idx = input.astype(jnp.int32).reshape(NW, NCHUNK, CHUNK)
    mesh = plsc.VectorSubcoreMesh(
        core_axis_name="c", subcore_axis_name="s", num_cores=NC, num_subcores=NS
    )
    run = functools.partial(
        pl.kernel,
        mesh=mesh,
        out_type=jax.ShapeDtypeStruct((TOKENS, DIM), jnp.float32),
        scratch_types=[
            pltpu.VMEM((NCHUNK, CHUNK), jnp.int32),
            pltpu.VMEM((GROUP * CHUNK, DIM), jnp.float32),
            pltpu.SemaphoreType.DMA,
        ],
        compiler_params=pltpu.CompilerParams(use_tc_tiling_on_sc=False),
    )(_body)
    out = run(idx, weight)
    return out.reshape(input.shape[0], input.shape[1], DIM)


# speedup vs baseline: 1.0007x; 1.0007x over previous
import functools

import jax
import jax.numpy as jnp
from jax import lax
from jax.experimental import pallas as pl
from jax.experimental.pallas import tpu as pltpu
from jax.experimental.pallas import tpu_sc as plsc

VOCAB = 1_000_000
DIM = 64
WORLD_SIZE = 1
RANK = 0
BLOCK = (VOCAB + WORLD_SIZE - 1) // WORLD_SIZE
MIN_ID = RANK * BLOCK
MAX_ID = min(VOCAB, (RANK + 1) * BLOCK)
NULL_IDX = MAX_ID - MIN_ID

NC = 2
NS = 16
NW = NC * NS
LANES = 16

TOKENS = 4096 * 50
BPW = TOKENS // NW
CHUNK = 128
NCHUNK = BPW // CHUNK
GROUP = 5
NGROUP = NCHUNK // GROUP


def _body(idx_hbm, w_hbm, out_hbm, idx_flat, idx_v, buf, sem):
    wid = lax.axis_index("s") * NC + lax.axis_index("c")
    pltpu.sync_copy(idx_hbm.at[pl.ds(wid * BPW, BPW)], idx_flat)

    def remap(t, carry):
        row = t // (CHUNK // LANES)
        col = (t % (CHUNK // LANES)) * LANES
        v = idx_flat[pl.ds(t * LANES, LANES)]
        oob = (v < MIN_ID) | (v >= MAX_ID)
        idx_v[row, pl.ds(col, LANES)] = jnp.where(oob, NULL_IDX, v - MIN_ID)
        return carry

    lax.fori_loop(0, BPW // LANES, remap, None)

    base = wid * BPW

    def group(g, carry):
        handles = []
        for b in range(GROUP):
            j = g * GROUP + b
            h = pltpu.async_copy(
                w_hbm.at[idx_v.at[j]], buf.at[pl.ds(b * CHUNK, CHUNK)], sem
            )
            handles.append(h)
        for h in handles:
            h.wait()
        pltpu.sync_copy(
            buf, out_hbm.at[pl.ds(base + g * (GROUP * CHUNK), GROUP * CHUNK)]
        )
        return carry

    lax.fori_loop(0, NGROUP, group, None)


@jax.jit
def kernel(input, weight):
    idx = input.astype(jnp.int32).reshape(TOKENS)
    mesh = plsc.VectorSubcoreMesh(
        core_axis_name="c", subcore_axis_name="s", num_cores=NC, num_subcores=NS
    )
    run = functools.partial(
        pl.kernel,
        mesh=mesh,
        out_type=jax.ShapeDtypeStruct((TOKENS, DIM), jnp.float32),
        scratch_types=[
            pltpu.VMEM((BPW,), jnp.int32),
            pltpu.VMEM((NCHUNK, CHUNK), jnp.int32),
            pltpu.VMEM((GROUP * CHUNK, DIM), jnp.float32),
            pltpu.SemaphoreType.DMA,
        ],
        compiler_params=pltpu.CompilerParams(use_tc_tiling_on_sc=False),
    )(_body)
    out = run(idx, weight)
    return out.reshape(input.shape[0], input.shape[1], DIM)


# speedup vs baseline: 1.0381x; 1.0373x over previous
import functools

import jax
import jax.numpy as jnp
from jax import lax
from jax.experimental import pallas as pl
from jax.experimental.pallas import tpu as pltpu
from jax.experimental.pallas import tpu_sc as plsc

VOCAB = 1_000_000
DIM = 64
WORLD_SIZE = 1
RANK = 0
BLOCK = (VOCAB + WORLD_SIZE - 1) // WORLD_SIZE
MIN_ID = RANK * BLOCK
MAX_ID = min(VOCAB, (RANK + 1) * BLOCK)
NULL_IDX = MAX_ID - MIN_ID

NC = 2
NS = 16
NW = NC * NS
LANES = 16

ROWS_PAD = 1_000_008
DIM_PAD = 128

TOKENS = 4096 * 50
BPW = TOKENS // NW
CHUNK = 128
NCHUNK = BPW // CHUNK
GROUP = 5
NGROUP = NCHUNK // GROUP


def _body(idx_hbm, w_hbm, out_hbm, idx_flat, idx_v, buf, sem):
    wid = lax.axis_index("s") * NC + lax.axis_index("c")
    pltpu.sync_copy(idx_hbm.at[pl.ds(wid * BPW, BPW)], idx_flat)

    def remap(t, carry):
        row = t // (CHUNK // LANES)
        col = (t % (CHUNK // LANES)) * LANES
        v = idx_flat[pl.ds(t * LANES, LANES)]
        oob = (v < MIN_ID) | (v >= MAX_ID)
        idx_v[row, pl.ds(col, LANES)] = jnp.where(oob, NULL_IDX, v - MIN_ID)
        return carry

    lax.fori_loop(0, BPW // LANES, remap, None)

    base = wid * BPW

    def group(g, carry):
        handles = []
        for b in range(GROUP):
            j = g * GROUP + b
            h = pltpu.async_copy(
                w_hbm.at[idx_v.at[j]], buf.at[pl.ds(b * CHUNK, CHUNK)], sem
            )
            handles.append(h)
        for h in handles:
            h.wait()
        pltpu.sync_copy(
            buf.at[:, pl.ds(0, DIM)],
            out_hbm.at[pl.ds(base + g * (GROUP * CHUNK), GROUP * CHUNK)],
        )
        return carry

    lax.fori_loop(0, NGROUP, group, None)


@jax.jit
def kernel(input, weight):
    idx = input.astype(jnp.int32).reshape(TOKENS)
    wpad = jnp.pad(weight, ((0, ROWS_PAD - (NULL_IDX + 1)), (0, DIM_PAD - DIM)))
    mesh = plsc.VectorSubcoreMesh(
        core_axis_name="c", subcore_axis_name="s", num_cores=NC, num_subcores=NS
    )
    run = functools.partial(
        pl.kernel,
        mesh=mesh,
        out_type=jax.ShapeDtypeStruct((TOKENS, DIM), jnp.float32),
        scratch_types=[
            pltpu.VMEM((BPW,), jnp.int32),
            pltpu.VMEM((NCHUNK, CHUNK), jnp.int32),
            pltpu.VMEM((GROUP * CHUNK, DIM_PAD), jnp.float32),
            pltpu.SemaphoreType.DMA,
        ],
        compiler_params=pltpu.CompilerParams(use_tc_tiling_on_sc=False),
    )(_body)
    out = run(idx, wpad)
    return out.reshape(input.shape[0], input.shape[1], DIM)
